# baseline (speedup 1.0000x reference)
import jax
import jax.numpy as jnp
from jax import lax
from jax.experimental import pallas as pl
from jax.experimental.pallas import tpu as pltpu
from jax.experimental.pallas import tpu_sc as plsc

B = 16384
V = 100000
F = 64
W = 2 * F
L = 16
NC, NS = 2, 16
NW = NC * NS
BPW = B // NW
CHUNK = 128
N_CHUNKS = BPW // CHUNK
N_BLOCKS = CHUNK // L


def _sc_body(uid_hbm, iid_hbm, pu_hbm, qi_hbm, pred_hbm, feat_hbm,
             uid_v, iid_v, pu_c, qi_c, feat_c, pred_v, sem):
    wid = lax.axis_index("s") * NC + lax.axis_index("c")
    base = wid * BPW

    pltpu.sync_copy(uid_hbm.at[pl.ds(base, BPW)], uid_v)
    pltpu.sync_copy(iid_hbm.at[pl.ds(base, BPW)], iid_v)

    lanes = lax.iota(jnp.int32, L)

    for j in range(N_CHUNKS):
        sl = pl.ds(j * CHUNK, CHUNK)
        cp = pltpu.async_copy(pu_hbm.at[0].at[uid_v.at[sl]], pu_c, sem)
        cq = pltpu.async_copy(qi_hbm.at[0].at[iid_v.at[sl]], qi_c, sem)
        cp.wait()
        cq.wait()

        def blk_body(blk, _, j=j):
            acc16 = jnp.zeros((L,), jnp.float32)
            for r16 in range(L):
                r = blk * L + r16
                acc = None
                for c in range(F // L):
                    p = pu_c[r, pl.ds(c * L, L)]
                    q = qi_c[r, pl.ds(c * L, L)]
                    feat_c[r, pl.ds(c * L, L)] = p
                    feat_c[r, pl.ds(F + c * L, L)] = q
                    acc = p * q if acc is None else acc + p * q
                s = jnp.sum(acc)
                acc16 = jnp.where(lanes == r16, s, acc16)
            acc16 = jnp.minimum(jnp.maximum(acc16, 1.0), 5.0)
            pred_v[pl.ds(j * CHUNK + blk * L, L)] = acc16
            return 0

        lax.fori_loop(0, N_BLOCKS, blk_body, 0)
        pltpu.sync_copy(feat_c, feat_hbm.at[pl.ds(base + j * CHUNK, CHUNK)])

    pltpu.sync_copy(pred_v, pred_hbm.at[pl.ds(base, BPW)])


def _gather_combine(uid, iid, pu3, qi3):
    mesh = plsc.VectorSubcoreMesh(core_axis_name="c", subcore_axis_name="s")
    return pl.kernel(
        _sc_body,
        out_type=(
            jax.ShapeDtypeStruct((B,), jnp.float32),
            jax.ShapeDtypeStruct((B, W), jnp.float32),
        ),
        mesh=mesh,
        compiler_params=pltpu.CompilerParams(use_tc_tiling_on_sc=False,
                                             needs_layout_passes=False),
        scratch_types=[
            pltpu.VMEM((BPW,), jnp.int32),
            pltpu.VMEM((BPW,), jnp.int32),
            pltpu.VMEM((CHUNK, F), jnp.float32),
            pltpu.VMEM((CHUNK, F), jnp.float32),
            pltpu.VMEM((CHUNK, W), jnp.float32),
            pltpu.VMEM((BPW,), jnp.float32),
            pltpu.SemaphoreType.DMA,
        ],
    )(uid, iid, pu3, qi3)


@jax.jit
def _run(user_item, pu, qi):
    return _gather_combine(user_item[:, 0], user_item[:, 1],
                           pu.reshape(1, V, F), qi.reshape(1, V, F))


def kernel(user_item, pu, qi):
    return _run(user_item.astype(jnp.int32), pu, qi)
